# baseline (speedup 1.0000x reference)
import functools

import jax
import jax.numpy as jnp
from jax import lax
from jax.experimental import pallas as pl
from jax.experimental.pallas import tpu as pltpu
from jax.experimental.pallas import tpu_sc as plsc

N_ROWS = 1000000
HDIM = 32
B_DIM = 16384
S_DIM = 200
_NC, _NS = 2, 16
NW = _NC * _NS

_mesh = plsc.VectorSubcoreMesh(core_axis_name="c", subcore_axis_name="s")

N_COL = N_ROWS // 128
_A_EXTRA = N_COL - NW * (N_COL // NW)
_A_PAIRS = (N_COL // NW) // 2 + 1


@functools.partial(
    pl.kernel,
    mesh=_mesh,
    out_type=jax.ShapeDtypeStruct((N_ROWS // 4, 128), jnp.float32),
    scratch_types=(
        [pltpu.VMEM((32, 128), jnp.float32) for _ in range(2)]
        + [pltpu.VMEM((32, 128), jnp.float32) for _ in range(2)]
        + [pltpu.VMEM((32, 64), jnp.float32)]
        + [pltpu.VMEM((16, 128), jnp.float32)]
        + [pltpu.SemaphoreType.DMA for _ in range(4)]
    ),
    compiler_params=pltpu.CompilerParams(
        use_tc_tiling_on_sc=True, needs_layout_passes=False),
)
def _repack(embsT_hbm, lin_hbm, *scratch):
    src_v = scratch[0:2]
    dst_v = scratch[2:4]
    tsrc_v = scratch[4]
    tdst_v = scratch[5]
    sem_r = scratch[6:8]
    sem_w = scratch[8:10]

    wid = lax.axis_index("s") * _NC + lax.axis_index("c")
    cnt = (N_COL // NW) + jnp.where(wid < _A_EXTRA, 1, 0)
    base = (N_COL // NW) * wid + jnp.minimum(wid, _A_EXTRA)
    iot = lax.iota(jnp.int32, 16)

    def start_read(k, b):
        pltpu.async_copy(
            embsT_hbm.at[:, pl.ds((base + k) * 128, 128)], src_v[b], sem_r[b])

    def wait_read(b):
        pltpu.make_async_copy(
            embsT_hbm.at[:, pl.ds(0, 128)], src_v[b], sem_r[b]).wait()

    def start_write(k, b):
        pltpu.async_copy(
            dst_v[b], lin_hbm.at[pl.ds((base + k) * 32, 32), :], sem_w[b])

    def wait_write(b):
        pltpu.make_async_copy(
            dst_v[b], lin_hbm.at[pl.ds(0, 32), :], sem_w[b]).wait()

    def transpose_block(src, dst, n_rr):
        for m in range(2 * n_rr):
            val = plsc.load_gather(
                src, [16 * (m % 2) + iot, jnp.full((16,), m // 2, jnp.int32)])
            dst[m // 8, pl.ds(16 * (m % 8), 16)] = val

    start_read(0, 0)
    start_read(1, 1)

    @pl.loop(0, _A_PAIRS)
    def _pair(kk):
        for b in range(2):
            k = kk * 2 + b

            @pl.when(k < cnt)
            def _():
                wait_read(b)

                @pl.when(k >= 2)
                def _():
                    wait_write(b)

                transpose_block(src_v[b], dst_v[b], 128)

                @pl.when(k + 2 < cnt)
                def _():
                    start_read(k + 2, b)

                start_write(k, b)

    wait_write(0)
    wait_write(1)

    @pl.when(wid == 0)
    def _tail():
        tsem = sem_r[0]
        pltpu.async_copy(
            embsT_hbm.at[:, pl.ds(N_COL * 128, 64)], tsrc_v, tsem).wait()
        transpose_block(tsrc_v, tdst_v, 64)
        pltpu.async_copy(
            tdst_v, lin_hbm.at[pl.ds(N_COL * 32, 16), :], tsem).wait()


CHUNK = B_DIM // NW
_SBLK = HDIM * B_DIM


@functools.partial(
    pl.kernel,
    mesh=_mesh,
    out_type=jax.ShapeDtypeStruct((S_DIM, _SBLK), jnp.float32),
    scratch_types=(
        [pltpu.VMEM((CHUNK,), jnp.int32) for _ in range(2)]
        + [pltpu.VMEM((CHUNK, HDIM), jnp.float32) for _ in range(2)]
        + [pltpu.VMEM((4, 4096), jnp.float32) for _ in range(2)]
        + [pltpu.SemaphoreType.DMA for _ in range(6)]
    ),
    compiler_params=pltpu.CompilerParams(
        use_tc_tiling_on_sc=False, needs_layout_passes=False),
)
def _gather(idx_hbm, table_hbm, out_hbm, *scratch):
    idx_v = scratch[0:2]
    rows_v = scratch[2:4]
    rowsT_v = scratch[4:6]
    sem_i = scratch[6:8]
    sem_g = scratch[8:10]
    sem_s = scratch[10:12]

    wid = lax.axis_index("s") * _NC + lax.axis_index("c")
    b0 = wid * CHUNK
    iot = lax.iota(jnp.int32, 16)

    def start_idx(s, b):
        pltpu.async_copy(
            idx_hbm.at[pl.ds(s * B_DIM + b0, CHUNK)], idx_v[b], sem_i[b])

    def wait_idx(b):
        pltpu.make_async_copy(
            idx_hbm.at[pl.ds(0, CHUNK)], idx_v[b], sem_i[b]).wait()

    def start_gather(b):
        pltpu.async_copy(table_hbm.at[idx_v[b]], rows_v[b], sem_g[b])

    def wait_gather(b):
        pltpu.make_async_copy(
            table_hbm.at[pl.ds(0, CHUNK)], rows_v[b], sem_g[b]).wait()

    def start_store(s, b):
        for i in range(4):
            pltpu.async_copy(
                rowsT_v[b].at[pl.ds(i, 1), :],
                out_hbm.at[pl.ds(s, 1),
                           pl.ds(i * (_SBLK // 4) + wid * 4096, 4096)],
                sem_s[b])

    def wait_store(b):
        for i in range(4):
            pltpu.make_async_copy(
                rowsT_v[b].at[pl.ds(i, 1), :],
                out_hbm.at[pl.ds(0, 1), pl.ds(0, 4096)], sem_s[b]).wait()

    def transpose_chunk(b):
        for i in range(4):
            @pl.loop(0, 4)
            def _jj(jj):
                for r in range(8):
                    for n in range(8):
                        val = plsc.load_gather(
                            rows_v[b],
                            [128 * jj + 16 * n + iot,
                             jnp.full((16,), 8 * i + r, jnp.int32)])
                        rowsT_v[b][i, pl.ds(1024 * jj + 128 * r + 16 * n, 16)] = val

    start_idx(0, 0)
    wait_idx(0)
    start_gather(0)
    start_idx(1, 1)

    @pl.loop(0, S_DIM // 2)
    def _step(t):
        for b in range(2):
            s = t * 2 + b
            bn = 1 - b

            @pl.when(s + 1 < S_DIM)
            def _():
                wait_idx(bn)
                start_gather(bn)

            wait_gather(b)

            @pl.when(s + 2 < S_DIM)
            def _():
                start_idx(s + 2, b)

            @pl.when(s >= 2)
            def _():
                wait_store(b)

            transpose_chunk(b)
            start_store(s, b)

    wait_store(0)
    wait_store(1)


def kernel(indices, embs):
    idx_flat = jnp.transpose(indices).reshape(-1)
    embsT = jnp.transpose(embs)
    table = _repack(embsT).reshape(N_ROWS, HDIM)
    img = _gather(idx_flat, table)
    x = img.reshape(S_DIM, 4, 128, 8, 128)
    w = jnp.transpose(x, (2, 4, 0, 1, 3))
    return w.reshape(B_DIM, S_DIM, HDIM)


# speedup vs baseline: 1.4698x; 1.4698x over previous
import functools

import jax
import jax.numpy as jnp
from jax import lax
from jax.experimental import pallas as pl
from jax.experimental.pallas import tpu as pltpu
from jax.experimental.pallas import tpu_sc as plsc

N_ROWS = 1000000
HDIM = 32
B_DIM = 16384
S_DIM = 200
_NC, _NS = 2, 16
NW = _NC * _NS

_mesh = plsc.VectorSubcoreMesh(core_axis_name="c", subcore_axis_name="s")

N_COL = N_ROWS // 128
_A_EXTRA = N_COL - NW * (N_COL // NW)
_A_PAIRS = (N_COL // NW) // 2 + 1


@functools.partial(
    pl.kernel,
    mesh=_mesh,
    out_type=jax.ShapeDtypeStruct((N_ROWS // 4, 128), jnp.float32),
    scratch_types=(
        [pltpu.VMEM((32, 128), jnp.float32) for _ in range(2)]
        + [pltpu.VMEM((32, 128), jnp.float32) for _ in range(2)]
        + [pltpu.VMEM((32, 64), jnp.float32)]
        + [pltpu.VMEM((16, 128), jnp.float32)]
        + [pltpu.SemaphoreType.DMA for _ in range(4)]
    ),
    compiler_params=pltpu.CompilerParams(
        use_tc_tiling_on_sc=True, needs_layout_passes=False),
)
def _repack(embsT_hbm, lin_hbm, *scratch):
    src_v = scratch[0:2]
    dst_v = scratch[2:4]
    tsrc_v = scratch[4]
    tdst_v = scratch[5]
    sem_r = scratch[6:8]
    sem_w = scratch[8:10]

    wid = lax.axis_index("s") * _NC + lax.axis_index("c")
    cnt = (N_COL // NW) + jnp.where(wid < _A_EXTRA, 1, 0)
    base = (N_COL // NW) * wid + jnp.minimum(wid, _A_EXTRA)
    iot = lax.iota(jnp.int32, 16)

    def start_read(k, b):
        pltpu.async_copy(
            embsT_hbm.at[:, pl.ds((base + k) * 128, 128)], src_v[b], sem_r[b])

    def wait_read(b):
        pltpu.make_async_copy(
            embsT_hbm.at[:, pl.ds(0, 128)], src_v[b], sem_r[b]).wait()

    def start_write(k, b):
        pltpu.async_copy(
            dst_v[b], lin_hbm.at[pl.ds((base + k) * 32, 32), :], sem_w[b])

    def wait_write(b):
        pltpu.make_async_copy(
            dst_v[b], lin_hbm.at[pl.ds(0, 32), :], sem_w[b]).wait()

    def transpose_block(src, dst, n_rr):
        @plsc.parallel_loop(0, 2 * n_rr, unroll=8)
        def _m(m):
            val = plsc.load_gather(
                src, [16 * (m % 2) + iot, jnp.full((16,), m // 2, jnp.int32)])
            dst[m // 8, pl.ds(16 * (m % 8), 16)] = val

    start_read(0, 0)
    start_read(1, 1)

    @pl.loop(0, _A_PAIRS)
    def _pair(kk):
        for b in range(2):
            k = kk * 2 + b

            @pl.when(k < cnt)
            def _():
                wait_read(b)

                @pl.when(k >= 2)
                def _():
                    wait_write(b)

                transpose_block(src_v[b], dst_v[b], 128)

                @pl.when(k + 2 < cnt)
                def _():
                    start_read(k + 2, b)

                start_write(k, b)

    wait_write(0)
    wait_write(1)

    @pl.when(wid == 0)
    def _tail():
        tsem = sem_r[0]
        pltpu.async_copy(
            embsT_hbm.at[:, pl.ds(N_COL * 128, 64)], tsrc_v, tsem).wait()
        transpose_block(tsrc_v, tdst_v, 64)
        pltpu.async_copy(
            tdst_v, lin_hbm.at[pl.ds(N_COL * 32, 16), :], tsem).wait()


CHUNK = B_DIM // NW
_SBLK = HDIM * B_DIM


@functools.partial(
    pl.kernel,
    mesh=_mesh,
    out_type=jax.ShapeDtypeStruct((S_DIM, _SBLK), jnp.float32),
    scratch_types=(
        [pltpu.VMEM((CHUNK,), jnp.int32) for _ in range(2)]
        + [pltpu.VMEM((CHUNK, HDIM), jnp.float32) for _ in range(2)]
        + [pltpu.VMEM((4, 4096), jnp.float32) for _ in range(2)]
        + [pltpu.SemaphoreType.DMA for _ in range(6)]
    ),
    compiler_params=pltpu.CompilerParams(
        use_tc_tiling_on_sc=False, needs_layout_passes=False),
)
def _gather(idx_hbm, table_hbm, out_hbm, *scratch):
    idx_v = scratch[0:2]
    rows_v = scratch[2:4]
    rowsT_v = scratch[4:6]
    sem_i = scratch[6:8]
    sem_g = scratch[8:10]
    sem_s = scratch[10:12]

    wid = lax.axis_index("s") * _NC + lax.axis_index("c")
    b0 = wid * CHUNK
    iot = lax.iota(jnp.int32, 16)

    def start_idx(s, b):
        pltpu.async_copy(
            idx_hbm.at[pl.ds(s * B_DIM + b0, CHUNK)], idx_v[b], sem_i[b])

    def wait_idx(b):
        pltpu.make_async_copy(
            idx_hbm.at[pl.ds(0, CHUNK)], idx_v[b], sem_i[b]).wait()

    def start_gather(b):
        pltpu.async_copy(table_hbm.at[idx_v[b]], rows_v[b], sem_g[b])

    def wait_gather(b):
        pltpu.make_async_copy(
            table_hbm.at[pl.ds(0, CHUNK)], rows_v[b], sem_g[b]).wait()

    def start_store(s, b):
        for i in range(4):
            pltpu.async_copy(
                rowsT_v[b].at[pl.ds(i, 1), :],
                out_hbm.at[pl.ds(s, 1),
                           pl.ds(i * (_SBLK // 4) + wid * 4096, 4096)],
                sem_s[b])

    def wait_store(b):
        for i in range(4):
            pltpu.make_async_copy(
                rowsT_v[b].at[pl.ds(i, 1), :],
                out_hbm.at[pl.ds(0, 1), pl.ds(0, 4096)], sem_s[b]).wait()

    def transpose_chunk(b):
        @plsc.parallel_loop(0, 16, unroll=2)
        def _m(m):
            i = m // 4
            jj = m % 4
            for r in range(8):
                col = jnp.full((16,), 8 * i + r, jnp.int32)
                for n in range(8):
                    val = plsc.load_gather(
                        rows_v[b], [128 * jj + 16 * n + iot, col])
                    rowsT_v[b][i, pl.ds(1024 * jj + 128 * r + 16 * n, 16)] = val

    start_idx(0, 0)
    wait_idx(0)
    start_gather(0)
    start_idx(1, 1)

    @pl.loop(0, S_DIM // 2)
    def _step(t):
        for b in range(2):
            s = t * 2 + b
            bn = 1 - b

            @pl.when(s + 1 < S_DIM)
            def _():
                wait_idx(bn)
                start_gather(bn)

            wait_gather(b)

            @pl.when(s + 2 < S_DIM)
            def _():
                start_idx(s + 2, b)

            @pl.when(s >= 2)
            def _():
                wait_store(b)

            transpose_chunk(b)
            start_store(s, b)

    wait_store(0)
    wait_store(1)


def kernel(indices, embs):
    idx_flat = jnp.transpose(indices).reshape(-1)
    embsT = jnp.transpose(embs)
    table = _repack(embsT).reshape(N_ROWS, HDIM)
    img = _gather(idx_flat, table)
    x = img.reshape(S_DIM, 4, 128, 8, 128)
    w = jnp.transpose(x, (2, 4, 0, 1, 3))
    return w.reshape(B_DIM, S_DIM, HDIM)


# speedup vs baseline: 3.6400x; 2.4766x over previous
import functools

import jax
import jax.numpy as jnp
from jax import lax
from jax.experimental import pallas as pl
from jax.experimental.pallas import tpu as pltpu
from jax.experimental.pallas import tpu_sc as plsc

N_ROWS = 1000000
HDIM = 32
B_DIM = 16384
S_DIM = 200
_NC, _NS = 2, 16
NW = _NC * _NS

_mesh = plsc.VectorSubcoreMesh(core_axis_name="c", subcore_axis_name="s")

N_COL = N_ROWS // 128
_A_EXTRA = N_COL - NW * (N_COL // NW)
_A_PAIRS = (N_COL // NW) // 2 + 1


@functools.partial(
    pl.kernel,
    mesh=_mesh,
    out_type=jax.ShapeDtypeStruct((N_ROWS // 4, 128), jnp.float32),
    scratch_types=(
        [pltpu.VMEM((32, 128), jnp.float32) for _ in range(2)]
        + [pltpu.VMEM((32, 128), jnp.float32) for _ in range(2)]
        + [pltpu.VMEM((32, 64), jnp.float32)]
        + [pltpu.VMEM((16, 128), jnp.float32)]
        + [pltpu.SemaphoreType.DMA for _ in range(4)]
    ),
    compiler_params=pltpu.CompilerParams(
        use_tc_tiling_on_sc=True, needs_layout_passes=False),
)
def _repack(embsT_hbm, lin_hbm, *scratch):
    src_v = scratch[0:2]
    dst_v = scratch[2:4]
    tsrc_v = scratch[4]
    tdst_v = scratch[5]
    sem_r = scratch[6:8]
    sem_w = scratch[8:10]

    wid = lax.axis_index("s") * _NC + lax.axis_index("c")
    cnt = (N_COL // NW) + jnp.where(wid < _A_EXTRA, 1, 0)
    base = (N_COL // NW) * wid + jnp.minimum(wid, _A_EXTRA)
    iot = lax.iota(jnp.int32, 16)

    def start_read(k, b):
        pltpu.async_copy(
            embsT_hbm.at[:, pl.ds((base + k) * 128, 128)], src_v[b], sem_r[b])

    def wait_read(b):
        pltpu.make_async_copy(
            embsT_hbm.at[:, pl.ds(0, 128)], src_v[b], sem_r[b]).wait()

    def start_write(k, b):
        pltpu.async_copy(
            dst_v[b], lin_hbm.at[pl.ds((base + k) * 32, 32), :], sem_w[b])

    def wait_write(b):
        pltpu.make_async_copy(
            dst_v[b], lin_hbm.at[pl.ds(0, 32), :], sem_w[b]).wait()

    def transpose_block(src, dst, n_rr):
        @plsc.parallel_loop(0, 2 * n_rr, unroll=8)
        def _m(m):
            val = plsc.load_gather(
                src, [16 * (m % 2) + iot, jnp.full((16,), m // 2, jnp.int32)])
            dst[m // 8, pl.ds(16 * (m % 8), 16)] = val

    start_read(0, 0)
    start_read(1, 1)

    @pl.loop(0, _A_PAIRS)
    def _pair(kk):
        for b in range(2):
            k = kk * 2 + b

            @pl.when(k < cnt)
            def _():
                wait_read(b)

                @pl.when(k >= 2)
                def _():
                    wait_write(b)

                transpose_block(src_v[b], dst_v[b], 128)

                @pl.when(k + 2 < cnt)
                def _():
                    start_read(k + 2, b)

                start_write(k, b)

    wait_write(0)
    wait_write(1)

    @pl.when(wid == 0)
    def _tail():
        tsem = sem_r[0]
        pltpu.async_copy(
            embsT_hbm.at[:, pl.ds(N_COL * 128, 64)], tsrc_v, tsem).wait()
        transpose_block(tsrc_v, tdst_v, 64)
        pltpu.async_copy(
            tdst_v, lin_hbm.at[pl.ds(N_COL * 32, 16), :], tsem).wait()


CHUNK = B_DIM // NW
_SBLK = HDIM * B_DIM


@functools.partial(
    pl.kernel,
    mesh=_mesh,
    out_type=jax.ShapeDtypeStruct((S_DIM, _SBLK), jnp.float32),
    scratch_types=(
        [pltpu.VMEM((CHUNK,), jnp.int32) for _ in range(2)]
        + [pltpu.VMEM((CHUNK, HDIM), jnp.float32) for _ in range(2)]
        + [pltpu.VMEM((4, 4096), jnp.float32) for _ in range(2)]
        + [pltpu.SemaphoreType.DMA for _ in range(6)]
    ),
    compiler_params=pltpu.CompilerParams(
        use_tc_tiling_on_sc=False, needs_layout_passes=False),
)
def _gather(idx_hbm, table_hbm, out_hbm, *scratch):
    idx_v = scratch[0:2]
    rows_v = scratch[2:4]
    rowsT_v = scratch[4:6]
    sem_i = scratch[6:8]
    sem_g = scratch[8:10]
    sem_s = scratch[10:12]

    wid = lax.axis_index("s") * _NC + lax.axis_index("c")
    b0 = wid * CHUNK
    iot = lax.iota(jnp.int32, 16)

    def start_idx(s, b):
        pltpu.async_copy(
            idx_hbm.at[pl.ds(s * B_DIM + b0, CHUNK)], idx_v[b], sem_i[b])

    def wait_idx(b):
        pltpu.make_async_copy(
            idx_hbm.at[pl.ds(0, CHUNK)], idx_v[b], sem_i[b]).wait()

    def start_gather(b):
        pltpu.async_copy(table_hbm.at[idx_v[b]], rows_v[b], sem_g[b])

    def wait_gather(b):
        pltpu.make_async_copy(
            table_hbm.at[pl.ds(0, CHUNK)], rows_v[b], sem_g[b]).wait()

    def start_store(s, b):
        for i in range(4):
            pltpu.async_copy(
                rowsT_v[b].at[pl.ds(i, 1), :],
                out_hbm.at[pl.ds(s, 1),
                           pl.ds(i * (_SBLK // 4) + wid * 4096, 4096)],
                sem_s[b])

    def wait_store(b):
        for i in range(4):
            pltpu.make_async_copy(
                rowsT_v[b].at[pl.ds(i, 1), :],
                out_hbm.at[pl.ds(0, 1), pl.ds(0, 4096)], sem_s[b]).wait()

    def transpose_chunk(b):
        @plsc.parallel_loop(0, 32, unroll=2)
        def _q(q):
            soff = (q // 8) * 1024 + (q % 8) * 16
            row = 16 * q + iot
            for p in range(2):
                for k in range(16):
                    rot = (iot + k) & 15
                    col = 16 * p + rot
                    val = plsc.load_gather(rows_v[b], [row, col])
                    off = ((rot & 7) << 7) + iot + soff
                    plsc.store_scatter(rowsT_v[b], [col >> 3, off], val)

    start_idx(0, 0)
    wait_idx(0)
    start_gather(0)
    start_idx(1, 1)

    @pl.loop(0, S_DIM // 2)
    def _step(t):
        for b in range(2):
            s = t * 2 + b
            bn = 1 - b

            @pl.when(s + 1 < S_DIM)
            def _():
                wait_idx(bn)
                start_gather(bn)

            wait_gather(b)

            @pl.when(s + 2 < S_DIM)
            def _():
                start_idx(s + 2, b)

            @pl.when(s >= 2)
            def _():
                wait_store(b)

            transpose_chunk(b)
            start_store(s, b)

    wait_store(0)
    wait_store(1)


def kernel(indices, embs):
    idx_flat = jnp.transpose(indices).reshape(-1)
    embsT = jnp.transpose(embs)
    table = _repack(embsT).reshape(N_ROWS, HDIM)
    img = _gather(idx_flat, table)
    x = img.reshape(S_DIM, 4, 128, 8, 128)
    w = jnp.transpose(x, (2, 4, 0, 1, 3))
    return w.reshape(B_DIM, S_DIM, HDIM)


# speedup vs baseline: 4.3049x; 1.1827x over previous
import functools

import jax
import jax.numpy as jnp
from jax import lax
from jax.experimental import pallas as pl
from jax.experimental.pallas import tpu as pltpu
from jax.experimental.pallas import tpu_sc as plsc

N_ROWS = 1000000
HDIM = 32
B_DIM = 16384
S_DIM = 200
_NC, _NS = 2, 16
NW = _NC * _NS

_mesh = plsc.VectorSubcoreMesh(core_axis_name="c", subcore_axis_name="s")

N_COL = N_ROWS // 128
_A_EXTRA = N_COL - NW * (N_COL // NW)
_A_PAIRS = (N_COL // NW) // 2 + 1


@functools.partial(
    pl.kernel,
    mesh=_mesh,
    out_type=jax.ShapeDtypeStruct((N_ROWS // 4, 128), jnp.float32),
    scratch_types=(
        [pltpu.VMEM((32, 128), jnp.float32) for _ in range(2)]
        + [pltpu.VMEM((32, 128), jnp.float32) for _ in range(2)]
        + [pltpu.VMEM((32, 64), jnp.float32)]
        + [pltpu.VMEM((16, 128), jnp.float32)]
        + [pltpu.SemaphoreType.DMA for _ in range(4)]
    ),
    compiler_params=pltpu.CompilerParams(
        use_tc_tiling_on_sc=True, needs_layout_passes=False),
)
def _repack(embsT_hbm, lin_hbm, *scratch):
    src_v = scratch[0:2]
    dst_v = scratch[2:4]
    tsrc_v = scratch[4]
    tdst_v = scratch[5]
    sem_r = scratch[6:8]
    sem_w = scratch[8:10]

    wid = lax.axis_index("s") * _NC + lax.axis_index("c")
    cnt = (N_COL // NW) + jnp.where(wid < _A_EXTRA, 1, 0)
    base = (N_COL // NW) * wid + jnp.minimum(wid, _A_EXTRA)
    iot = lax.iota(jnp.int32, 16)

    def start_read(k, b):
        pltpu.async_copy(
            embsT_hbm.at[:, pl.ds((base + k) * 128, 128)], src_v[b], sem_r[b])

    def wait_read(b):
        pltpu.make_async_copy(
            embsT_hbm.at[:, pl.ds(0, 128)], src_v[b], sem_r[b]).wait()

    def start_write(k, b):
        pltpu.async_copy(
            dst_v[b], lin_hbm.at[pl.ds((base + k) * 32, 32), :], sem_w[b])

    def wait_write(b):
        pltpu.make_async_copy(
            dst_v[b], lin_hbm.at[pl.ds(0, 32), :], sem_w[b]).wait()

    def transpose_block(src, dst, n_rr):
        @plsc.parallel_loop(0, n_rr // 16, unroll=2)
        def _q(q):
            for p in range(2):
                drow = 16 * p + iot
                for k in range(16):
                    rot = (iot + k) & 15
                    col = 16 * q + rot
                    val = plsc.load_gather(src, [drow, col])
                    f = (col << 5) + 16 * p + iot
                    plsc.store_scatter(dst, [f >> 7, f & 127], val)

    start_read(0, 0)
    start_read(1, 1)

    @pl.loop(0, _A_PAIRS)
    def _pair(kk):
        for b in range(2):
            k = kk * 2 + b

            @pl.when(k < cnt)
            def _():
                wait_read(b)

                @pl.when(k >= 2)
                def _():
                    wait_write(b)

                transpose_block(src_v[b], dst_v[b], 128)

                @pl.when(k + 2 < cnt)
                def _():
                    start_read(k + 2, b)

                start_write(k, b)

    wait_write(0)
    wait_write(1)

    @pl.when(wid == 0)
    def _tail():
        tsem = sem_r[0]
        pltpu.async_copy(
            embsT_hbm.at[:, pl.ds(N_COL * 128, 64)], tsrc_v, tsem).wait()
        transpose_block(tsrc_v, tdst_v, 64)
        pltpu.async_copy(
            tdst_v, lin_hbm.at[pl.ds(N_COL * 32, 16), :], tsem).wait()


CHUNK = B_DIM // NW
_SBLK = HDIM * B_DIM


@functools.partial(
    pl.kernel,
    mesh=_mesh,
    out_type=jax.ShapeDtypeStruct((S_DIM, _SBLK), jnp.float32),
    scratch_types=(
        [pltpu.VMEM((CHUNK,), jnp.int32) for _ in range(2)]
        + [pltpu.VMEM((CHUNK, HDIM), jnp.float32) for _ in range(2)]
        + [pltpu.VMEM((4, 4096), jnp.float32) for _ in range(2)]
        + [pltpu.SemaphoreType.DMA for _ in range(6)]
    ),
    compiler_params=pltpu.CompilerParams(
        use_tc_tiling_on_sc=False, needs_layout_passes=False),
)
def _gather(idx_hbm, table_hbm, out_hbm, *scratch):
    idx_v = scratch[0:2]
    rows_v = scratch[2:4]
    rowsT_v = scratch[4:6]
    sem_i = scratch[6:8]
    sem_g = scratch[8:10]
    sem_s = scratch[10:12]

    wid = lax.axis_index("s") * _NC + lax.axis_index("c")
    b0 = wid * CHUNK
    iot = lax.iota(jnp.int32, 16)

    def start_idx(s, b):
        pltpu.async_copy(
            idx_hbm.at[pl.ds(s * B_DIM + b0, CHUNK)], idx_v[b], sem_i[b])

    def wait_idx(b):
        pltpu.make_async_copy(
            idx_hbm.at[pl.ds(0, CHUNK)], idx_v[b], sem_i[b]).wait()

    def start_gather(b):
        pltpu.async_copy(table_hbm.at[idx_v[b]], rows_v[b], sem_g[b])

    def wait_gather(b):
        pltpu.make_async_copy(
            table_hbm.at[pl.ds(0, CHUNK)], rows_v[b], sem_g[b]).wait()

    def start_store(s, b):
        for i in range(4):
            pltpu.async_copy(
                rowsT_v[b].at[pl.ds(i, 1), :],
                out_hbm.at[pl.ds(s, 1),
                           pl.ds(i * (_SBLK // 4) + wid * 4096, 4096)],
                sem_s[b])

    def wait_store(b):
        for i in range(4):
            pltpu.make_async_copy(
                rowsT_v[b].at[pl.ds(i, 1), :],
                out_hbm.at[pl.ds(0, 1), pl.ds(0, 4096)], sem_s[b]).wait()

    def transpose_chunk(b):
        @plsc.parallel_loop(0, 32, unroll=2)
        def _q(q):
            soff = (q // 8) * 1024 + (q % 8) * 16
            row = 16 * q + iot
            for p in range(2):
                for k in range(16):
                    rot = (iot + k) & 15
                    col = 16 * p + rot
                    val = plsc.load_gather(rows_v[b], [row, col])
                    off = ((rot & 7) << 7) + iot + soff
                    plsc.store_scatter(rowsT_v[b], [col >> 3, off], val)

    start_idx(0, 0)
    wait_idx(0)
    start_gather(0)
    start_idx(1, 1)

    @pl.loop(0, S_DIM // 2)
    def _step(t):
        for b in range(2):
            s = t * 2 + b
            bn = 1 - b

            @pl.when(s + 1 < S_DIM)
            def _():
                wait_idx(bn)
                start_gather(bn)

            wait_gather(b)

            @pl.when(s + 2 < S_DIM)
            def _():
                start_idx(s + 2, b)

            @pl.when(s >= 2)
            def _():
                wait_store(b)

            transpose_chunk(b)
            start_store(s, b)

    wait_store(0)
    wait_store(1)


def kernel(indices, embs):
    idx_flat = jnp.transpose(indices).reshape(-1)
    embsT = jnp.transpose(embs)
    table = _repack(embsT).reshape(N_ROWS, HDIM)
    img = _gather(idx_flat, table)
    x = img.reshape(S_DIM, 4, 128, 8, 128)
    w = jnp.transpose(x, (2, 4, 0, 1, 3))
    return w.reshape(B_DIM, S_DIM, HDIM)


# speedup vs baseline: 5.4470x; 1.2653x over previous
import functools

import jax
import jax.numpy as jnp
from jax import lax
from jax.experimental import pallas as pl
from jax.experimental.pallas import tpu as pltpu
from jax.experimental.pallas import tpu_sc as plsc

N_ROWS = 1000000
HDIM = 32
B_DIM = 16384
S_DIM = 200
_NC, _NS = 2, 16
NW = _NC * _NS

_mesh = plsc.VectorSubcoreMesh(core_axis_name="c", subcore_axis_name="s")

N_COL = N_ROWS // 256
_A_EXTRA = N_COL - NW * (N_COL // NW)
_A_PAIRS = (N_COL // NW) // 2 + 1


@functools.partial(
    pl.kernel,
    mesh=_mesh,
    out_type=jax.ShapeDtypeStruct((N_ROWS // 4, 128), jnp.float32),
    scratch_types=(
        [pltpu.VMEM((32, 256), jnp.float32) for _ in range(2)]
        + [pltpu.VMEM((64, 128), jnp.float32) for _ in range(2)]
        + [pltpu.VMEM((32, 64), jnp.float32)]
        + [pltpu.VMEM((16, 128), jnp.float32)]
        + [pltpu.SemaphoreType.DMA for _ in range(4)]
    ),
    compiler_params=pltpu.CompilerParams(
        use_tc_tiling_on_sc=True, needs_layout_passes=False),
)
def _repack(embsT_hbm, lin_hbm, *scratch):
    src_v = scratch[0:2]
    dst_v = scratch[2:4]
    tsrc_v = scratch[4]
    tdst_v = scratch[5]
    sem_r = scratch[6:8]
    sem_w = scratch[8:10]

    wid = lax.axis_index("s") * _NC + lax.axis_index("c")
    cnt = (N_COL // NW) + jnp.where(wid < _A_EXTRA, 1, 0)
    base = (N_COL // NW) * wid + jnp.minimum(wid, _A_EXTRA)
    iot = lax.iota(jnp.int32, 16)

    def start_read(k, b):
        pltpu.async_copy(
            embsT_hbm.at[:, pl.ds((base + k) * 256, 256)], src_v[b], sem_r[b])

    def wait_read(b):
        pltpu.make_async_copy(
            embsT_hbm.at[:, pl.ds(0, 256)], src_v[b], sem_r[b]).wait()

    def start_write(k, b):
        pltpu.async_copy(
            dst_v[b], lin_hbm.at[pl.ds((base + k) * 64, 64), :], sem_w[b])

    def wait_write(b):
        pltpu.make_async_copy(
            dst_v[b], lin_hbm.at[pl.ds(0, 64), :], sem_w[b]).wait()

    def transpose_block(src, dst, n_rr):
        @plsc.parallel_loop(0, n_rr // 16, unroll=4)
        def _q(q):
            for p in range(2):
                drow = 16 * p + iot
                for k in range(16):
                    rot = (iot + k) & 15
                    col = 16 * q + rot
                    val = plsc.load_gather(src, [drow, col])
                    f = (col << 5) + 16 * p + iot
                    plsc.store_scatter(dst, [f >> 7, f & 127], val)

    start_read(0, 0)
    start_read(1, 1)

    @pl.loop(0, _A_PAIRS)
    def _pair(kk):
        for b in range(2):
            k = kk * 2 + b

            @pl.when(k < cnt)
            def _():
                wait_read(b)

                @pl.when(k >= 2)
                def _():
                    wait_write(b)

                transpose_block(src_v[b], dst_v[b], 256)

                @pl.when(k + 2 < cnt)
                def _():
                    start_read(k + 2, b)

                start_write(k, b)

    wait_write(0)
    wait_write(1)

    @pl.when(wid == 0)
    def _tail():
        tsem = sem_r[0]
        pltpu.async_copy(
            embsT_hbm.at[:, pl.ds(N_COL * 256, 64)], tsrc_v, tsem).wait()
        transpose_block(tsrc_v, tdst_v, 64)
        pltpu.async_copy(
            tdst_v, lin_hbm.at[pl.ds(N_COL * 64, 16), :], tsem).wait()


CHUNK = B_DIM // NW
_SBLK = HDIM * B_DIM


@functools.partial(
    pl.kernel,
    mesh=_mesh,
    out_type=jax.ShapeDtypeStruct((S_DIM, _SBLK), jnp.float32),
    scratch_types=(
        [pltpu.VMEM((CHUNK,), jnp.int32) for _ in range(2)]
        + [pltpu.VMEM((CHUNK, HDIM), jnp.float32) for _ in range(2)]
        + [pltpu.VMEM((4, 4096), jnp.float32) for _ in range(2)]
        + [pltpu.SemaphoreType.DMA for _ in range(6)]
    ),
    compiler_params=pltpu.CompilerParams(
        use_tc_tiling_on_sc=False, needs_layout_passes=False),
)
def _gather(idx_hbm, table_hbm, out_hbm, *scratch):
    idx_v = scratch[0:2]
    rows_v = scratch[2:4]
    rowsT_v = scratch[4:6]
    sem_i = scratch[6:8]
    sem_g = scratch[8:10]
    sem_s = scratch[10:12]

    wid = lax.axis_index("s") * _NC + lax.axis_index("c")
    b0 = wid * CHUNK
    iot = lax.iota(jnp.int32, 16)

    def start_idx(s, b):
        pltpu.async_copy(
            idx_hbm.at[pl.ds(s * B_DIM + b0, CHUNK)], idx_v[b], sem_i[b])

    def wait_idx(b):
        pltpu.make_async_copy(
            idx_hbm.at[pl.ds(0, CHUNK)], idx_v[b], sem_i[b]).wait()

    def start_gather(b):
        pltpu.async_copy(table_hbm.at[idx_v[b]], rows_v[b], sem_g[b])

    def wait_gather(b):
        pltpu.make_async_copy(
            table_hbm.at[pl.ds(0, CHUNK)], rows_v[b], sem_g[b]).wait()

    def start_store(s, b):
        for i in range(4):
            pltpu.async_copy(
                rowsT_v[b].at[pl.ds(i, 1), :],
                out_hbm.at[pl.ds(s, 1),
                           pl.ds(i * (_SBLK // 4) + wid * 4096, 4096)],
                sem_s[b])

    def wait_store(b):
        for i in range(4):
            pltpu.make_async_copy(
                rowsT_v[b].at[pl.ds(i, 1), :],
                out_hbm.at[pl.ds(0, 1), pl.ds(0, 4096)], sem_s[b]).wait()

    def transpose_chunk(b):
        @plsc.parallel_loop(0, 32, unroll=4)
        def _q(q):
            soff = (q // 8) * 1024 + (q % 8) * 16
            row = 16 * q + iot
            for p in range(2):
                for k in range(16):
                    rot = (iot + k) & 15
                    col = 16 * p + rot
                    val = plsc.load_gather(rows_v[b], [row, col])
                    off = ((rot & 7) << 7) + iot + soff
                    plsc.store_scatter(rowsT_v[b], [col >> 3, off], val)

    start_idx(0, 0)
    wait_idx(0)
    start_gather(0)
    start_idx(1, 1)

    @pl.loop(0, S_DIM // 2)
    def _step(t):
        for b in range(2):
            s = t * 2 + b
            bn = 1 - b

            @pl.when(s + 1 < S_DIM)
            def _():
                wait_idx(bn)
                start_gather(bn)

            wait_gather(b)

            @pl.when(s + 2 < S_DIM)
            def _():
                start_idx(s + 2, b)

            @pl.when(s >= 2)
            def _():
                wait_store(b)

            transpose_chunk(b)
            start_store(s, b)

    wait_store(0)
    wait_store(1)


def kernel(indices, embs):
    idx_flat = jnp.transpose(indices).reshape(-1)
    embsT = jnp.transpose(embs)
    table = _repack(embsT).reshape(N_ROWS, HDIM)
    img = _gather(idx_flat, table)
    x = img.reshape(S_DIM, 4, 128, 8, 128)
    w = jnp.transpose(x, (2, 4, 0, 1, 3))
    return w.reshape(B_DIM, S_DIM, HDIM)


# speedup vs baseline: 6.0396x; 1.1088x over previous
import functools

import jax
import jax.numpy as jnp
from jax import lax
from jax.experimental import pallas as pl
from jax.experimental.pallas import tpu as pltpu
from jax.experimental.pallas import tpu_sc as plsc

N_ROWS = 1000000
HDIM = 32
B_DIM = 16384
S_DIM = 200
_NC, _NS = 2, 16
NW = _NC * _NS

_mesh = plsc.VectorSubcoreMesh(core_axis_name="c", subcore_axis_name="s")

N_COL = N_ROWS // 512
_A_EXTRA = N_COL - NW * (N_COL // NW)
_A_PAIRS = (N_COL // NW) // 2 + 1


@functools.partial(
    pl.kernel,
    mesh=_mesh,
    out_type=jax.ShapeDtypeStruct((N_ROWS // 4, 128), jnp.float32),
    scratch_types=(
        [pltpu.VMEM((32, 512), jnp.float32) for _ in range(2)]
        + [pltpu.VMEM((128, 128), jnp.float32) for _ in range(2)]
        + [pltpu.VMEM((32, 64), jnp.float32)]
        + [pltpu.VMEM((16, 128), jnp.float32)]
        + [pltpu.SemaphoreType.DMA for _ in range(4)]
    ),
    compiler_params=pltpu.CompilerParams(
        use_tc_tiling_on_sc=True, needs_layout_passes=False),
)
def _repack(embsT_hbm, lin_hbm, *scratch):
    src_v = scratch[0:2]
    dst_v = scratch[2:4]
    tsrc_v = scratch[4]
    tdst_v = scratch[5]
    sem_r = scratch[6:8]
    sem_w = scratch[8:10]

    wid = lax.axis_index("s") * _NC + lax.axis_index("c")
    cnt = (N_COL // NW) + jnp.where(wid < _A_EXTRA, 1, 0)
    base = (N_COL // NW) * wid + jnp.minimum(wid, _A_EXTRA)
    iot = lax.iota(jnp.int32, 16)

    def start_read(k, b):
        pltpu.async_copy(
            embsT_hbm.at[:, pl.ds((base + k) * 512, 512)], src_v[b], sem_r[b])

    def wait_read(b):
        pltpu.make_async_copy(
            embsT_hbm.at[:, pl.ds(0, 512)], src_v[b], sem_r[b]).wait()

    def start_write(k, b):
        pltpu.async_copy(
            dst_v[b], lin_hbm.at[pl.ds((base + k) * 128, 128), :], sem_w[b])

    def wait_write(b):
        pltpu.make_async_copy(
            dst_v[b], lin_hbm.at[pl.ds(0, 128), :], sem_w[b]).wait()

    def transpose_block(src, dst, n_rr):
        @plsc.parallel_loop(0, n_rr // 16, unroll=4)
        def _q(q):
            for p in range(2):
                drow = 16 * p + iot
                for k in range(16):
                    rot = (iot + k) & 15
                    col = 16 * q + rot
                    val = plsc.load_gather(src, [drow, col])
                    f = (col << 5) + 16 * p + iot
                    plsc.store_scatter(dst, [f >> 7, f & 127], val)

    start_read(0, 0)
    start_read(1, 1)

    @pl.loop(0, _A_PAIRS)
    def _pair(kk):
        for b in range(2):
            k = kk * 2 + b

            @pl.when(k < cnt)
            def _():
                wait_read(b)

                @pl.when(k >= 2)
                def _():
                    wait_write(b)

                transpose_block(src_v[b], dst_v[b], 512)

                @pl.when(k + 2 < cnt)
                def _():
                    start_read(k + 2, b)

                start_write(k, b)

    wait_write(0)
    wait_write(1)

    @pl.when(wid == 0)
    def _tail():
        tsem = sem_r[0]
        pltpu.async_copy(
            embsT_hbm.at[:, pl.ds(N_COL * 512, 64)], tsrc_v, tsem).wait()
        transpose_block(tsrc_v, tdst_v, 64)
        pltpu.async_copy(
            tdst_v, lin_hbm.at[pl.ds(N_COL * 128, 16), :], tsem).wait()


CHUNK = B_DIM // NW
_SBLK = HDIM * B_DIM


@functools.partial(
    pl.kernel,
    mesh=_mesh,
    out_type=jax.ShapeDtypeStruct((S_DIM, _SBLK), jnp.float32),
    scratch_types=(
        [pltpu.VMEM((CHUNK,), jnp.int32) for _ in range(2)]
        + [pltpu.VMEM((CHUNK, HDIM), jnp.float32) for _ in range(2)]
        + [pltpu.VMEM((4, 4096), jnp.float32) for _ in range(2)]
        + [pltpu.SemaphoreType.DMA for _ in range(6)]
    ),
    compiler_params=pltpu.CompilerParams(
        use_tc_tiling_on_sc=False, needs_layout_passes=False),
)
def _gather(idx_hbm, table_hbm, out_hbm, *scratch):
    idx_v = scratch[0:2]
    rows_v = scratch[2:4]
    rowsT_v = scratch[4:6]
    sem_i = scratch[6:8]
    sem_g = scratch[8:10]
    sem_s = scratch[10:12]

    wid = lax.axis_index("s") * _NC + lax.axis_index("c")
    b0 = wid * CHUNK
    iot = lax.iota(jnp.int32, 16)

    def start_idx(s, b):
        pltpu.async_copy(
            idx_hbm.at[pl.ds(s * B_DIM + b0, CHUNK)], idx_v[b], sem_i[b])

    def wait_idx(b):
        pltpu.make_async_copy(
            idx_hbm.at[pl.ds(0, CHUNK)], idx_v[b], sem_i[b]).wait()

    def start_gather(b):
        pltpu.async_copy(table_hbm.at[idx_v[b]], rows_v[b], sem_g[b])

    def wait_gather(b):
        pltpu.make_async_copy(
            table_hbm.at[pl.ds(0, CHUNK)], rows_v[b], sem_g[b]).wait()

    def start_store(s, b):
        for i in range(4):
            pltpu.async_copy(
                rowsT_v[b].at[pl.ds(i, 1), :],
                out_hbm.at[pl.ds(s, 1),
                           pl.ds(i * (_SBLK // 4) + wid * 4096, 4096)],
                sem_s[b])

    def wait_store(b):
        for i in range(4):
            pltpu.make_async_copy(
                rowsT_v[b].at[pl.ds(i, 1), :],
                out_hbm.at[pl.ds(0, 1), pl.ds(0, 4096)], sem_s[b]).wait()

    def transpose_chunk(b):
        @plsc.parallel_loop(0, 32, unroll=4)
        def _q(q):
            soff = (q // 8) * 1024 + (q % 8) * 16
            row = 16 * q + iot
            for p in range(2):
                for k in range(16):
                    rot = (iot + k) & 15
                    col = 16 * p + rot
                    val = plsc.load_gather(rows_v[b], [row, col])
                    off = ((rot & 7) << 7) + iot + soff
                    plsc.store_scatter(rowsT_v[b], [col >> 3, off], val)

    start_idx(0, 0)
    wait_idx(0)
    start_gather(0)
    start_idx(1, 1)

    @pl.loop(0, S_DIM // 2)
    def _step(t):
        for b in range(2):
            s = t * 2 + b
            bn = 1 - b

            @pl.when(s + 1 < S_DIM)
            def _():
                wait_idx(bn)
                start_gather(bn)

            wait_gather(b)

            @pl.when(s + 2 < S_DIM)
            def _():
                start_idx(s + 2, b)

            @pl.when(s >= 2)
            def _():
                wait_store(b)

            transpose_chunk(b)
            start_store(s, b)

    wait_store(0)
    wait_store(1)


def kernel(indices, embs):
    idx_flat = jnp.transpose(indices).reshape(-1)
    embsT = jnp.transpose(embs)
    table = _repack(embsT).reshape(N_ROWS, HDIM)
    img = _gather(idx_flat, table)
    x = img.reshape(S_DIM, 4, 128, 8, 128)
    w = jnp.transpose(x, (2, 4, 0, 1, 3))
    return w.reshape(B_DIM, S_DIM, HDIM)


# speedup vs baseline: 6.0703x; 1.0051x over previous
import functools

import jax
import jax.numpy as jnp
from jax import lax
from jax.experimental import pallas as pl
from jax.experimental.pallas import tpu as pltpu
from jax.experimental.pallas import tpu_sc as plsc

N_ROWS = 1000000
HDIM = 32
B_DIM = 16384
S_DIM = 200
_NC, _NS = 2, 16
NW = _NC * _NS

_mesh = plsc.VectorSubcoreMesh(core_axis_name="c", subcore_axis_name="s")

N_COL = N_ROWS // 512
_A_EXTRA = N_COL - NW * (N_COL // NW)
_A_PAIRS = (N_COL // NW) // 2 + 1


@functools.partial(
    pl.kernel,
    mesh=_mesh,
    out_type=jax.ShapeDtypeStruct((N_ROWS // 4, 128), jnp.float32),
    scratch_types=(
        [pltpu.VMEM((32, 512), jnp.float32) for _ in range(2)]
        + [pltpu.VMEM((128, 128), jnp.float32) for _ in range(2)]
        + [pltpu.VMEM((32, 64), jnp.float32)]
        + [pltpu.VMEM((16, 128), jnp.float32)]
        + [pltpu.SemaphoreType.DMA for _ in range(4)]
    ),
    compiler_params=pltpu.CompilerParams(
        use_tc_tiling_on_sc=True, needs_layout_passes=False),
)
def _repack(embsT_hbm, lin_hbm, *scratch):
    src_v = scratch[0:2]
    dst_v = scratch[2:4]
    tsrc_v = scratch[4]
    tdst_v = scratch[5]
    sem_r = scratch[6:8]
    sem_w = scratch[8:10]

    wid = lax.axis_index("s") * _NC + lax.axis_index("c")
    cnt = (N_COL // NW) + jnp.where(wid < _A_EXTRA, 1, 0)
    base = (N_COL // NW) * wid + jnp.minimum(wid, _A_EXTRA)
    iot = lax.iota(jnp.int32, 16)

    def start_read(k, b):
        pltpu.async_copy(
            embsT_hbm.at[:, pl.ds((base + k) * 512, 512)], src_v[b], sem_r[b])

    def wait_read(b):
        pltpu.make_async_copy(
            embsT_hbm.at[:, pl.ds(0, 512)], src_v[b], sem_r[b]).wait()

    def start_write(k, b):
        pltpu.async_copy(
            dst_v[b], lin_hbm.at[pl.ds((base + k) * 128, 128), :], sem_w[b])

    def wait_write(b):
        pltpu.make_async_copy(
            dst_v[b], lin_hbm.at[pl.ds(0, 128), :], sem_w[b]).wait()

    def transpose_block(src, dst, n_rr):
        @plsc.parallel_loop(0, n_rr // 16, unroll=4)
        def _q(q):
            for p in range(2):
                drow = 16 * p + iot
                for k in range(16):
                    rot = (iot + k) & 15
                    col = 16 * q + rot
                    val = plsc.load_gather(src, [drow, col])
                    f = (col << 5) + 16 * p + iot
                    plsc.store_scatter(dst, [f >> 7, f & 127], val)

    start_read(0, 0)
    start_read(1, 1)

    @pl.loop(0, _A_PAIRS)
    def _pair(kk):
        for b in range(2):
            k = kk * 2 + b

            @pl.when(k < cnt)
            def _():
                wait_read(b)

                @pl.when(k >= 2)
                def _():
                    wait_write(b)

                transpose_block(src_v[b], dst_v[b], 512)

                @pl.when(k + 2 < cnt)
                def _():
                    start_read(k + 2, b)

                start_write(k, b)

    wait_write(0)
    wait_write(1)

    @pl.when(wid == 0)
    def _tail():
        tsem = sem_r[0]
        pltpu.async_copy(
            embsT_hbm.at[:, pl.ds(N_COL * 512, 64)], tsrc_v, tsem).wait()
        transpose_block(tsrc_v, tdst_v, 64)
        pltpu.async_copy(
            tdst_v, lin_hbm.at[pl.ds(N_COL * 128, 16), :], tsem).wait()


CHUNK = B_DIM // NW
_SBLK = HDIM * B_DIM


@functools.partial(
    pl.kernel,
    mesh=_mesh,
    out_type=jax.ShapeDtypeStruct((S_DIM, _SBLK), jnp.float32),
    scratch_types=(
        [pltpu.VMEM((CHUNK,), jnp.int32) for _ in range(3)]
        + [pltpu.VMEM((CHUNK, HDIM), jnp.float32) for _ in range(3)]
        + [pltpu.VMEM((4, 4096), jnp.float32) for _ in range(3)]
        + [pltpu.SemaphoreType.DMA for _ in range(9)]
    ),
    compiler_params=pltpu.CompilerParams(
        use_tc_tiling_on_sc=False, needs_layout_passes=False),
)
def _gather(idx_hbm, table_hbm, out_hbm, *scratch):
    idx_v = scratch[0:3]
    rows_v = scratch[3:6]
    rowsT_v = scratch[6:9]
    sem_i = scratch[9:12]
    sem_g = scratch[12:15]
    sem_s = scratch[15:18]

    wid = lax.axis_index("s") * _NC + lax.axis_index("c")
    b0 = wid * CHUNK
    iot = lax.iota(jnp.int32, 16)

    def start_idx(s, b):
        pltpu.async_copy(
            idx_hbm.at[pl.ds(s * B_DIM + b0, CHUNK)], idx_v[b], sem_i[b])

    def wait_idx(b):
        pltpu.make_async_copy(
            idx_hbm.at[pl.ds(0, CHUNK)], idx_v[b], sem_i[b]).wait()

    def start_gather(b):
        pltpu.async_copy(table_hbm.at[idx_v[b]], rows_v[b], sem_g[b])

    def wait_gather(b):
        pltpu.make_async_copy(
            table_hbm.at[pl.ds(0, CHUNK)], rows_v[b], sem_g[b]).wait()

    def start_store(s, b):
        for i in range(4):
            pltpu.async_copy(
                rowsT_v[b].at[pl.ds(i, 1), :],
                out_hbm.at[pl.ds(s, 1),
                           pl.ds(i * (_SBLK // 4) + wid * 4096, 4096)],
                sem_s[b])

    def wait_store(b):
        for i in range(4):
            pltpu.make_async_copy(
                rowsT_v[b].at[pl.ds(i, 1), :],
                out_hbm.at[pl.ds(0, 1), pl.ds(0, 4096)], sem_s[b]).wait()

    def transpose_chunk(b):
        @plsc.parallel_loop(0, 32, unroll=4)
        def _q(q):
            soff = (q // 8) * 1024 + (q % 8) * 16
            row = 16 * q + iot
            for p in range(2):
                for k in range(16):
                    rot = (iot + k) & 15
                    col = 16 * p + rot
                    val = plsc.load_gather(rows_v[b], [row, col])
                    off = ((rot & 7) << 7) + iot + soff
                    plsc.store_scatter(rowsT_v[b], [col >> 3, off], val)

    start_idx(0, 0)
    start_idx(1, 1)
    wait_idx(0)
    start_gather(0)
    wait_idx(1)
    start_gather(1)
    start_idx(2, 2)

    @pl.loop(0, (S_DIM + 2) // 3)
    def _step(t):
        for b in range(3):
            s = t * 3 + b

            @pl.when(s < S_DIM)
            def _():
                bn = (b + 2) % 3

                @pl.when(s + 2 < S_DIM)
                def _():
                    wait_idx(bn)
                    start_gather(bn)

                wait_gather(b)

                @pl.when(s + 3 < S_DIM)
                def _():
                    start_idx(s + 3, b)

                @pl.when(s >= 3)
                def _():
                    wait_store(b)

                transpose_chunk(b)
                start_store(s, b)

    wait_store(0)
    wait_store(1)
    wait_store(2)


def kernel(indices, embs):
    idx_flat = jnp.transpose(indices).reshape(-1)
    embsT = jnp.transpose(embs)
    table = _repack(embsT).reshape(N_ROWS, HDIM)
    img = _gather(idx_flat, table)
    x = img.reshape(S_DIM, 4, 128, 8, 128)
    w = jnp.transpose(x, (2, 4, 0, 1, 3))
    return w.reshape(B_DIM, S_DIM, HDIM)
